# baseline (speedup 1.0000x reference)
import functools

import jax
import jax.numpy as jnp
from jax import lax
from jax.experimental import pallas as pl
from jax.experimental.pallas import tpu as pltpu
from jax.experimental.pallas import tpu_sc as plsc

HIDDEN = 768
NUM_EMB = 32
VOCAB = 8192
SEQ = 4096
V_BLK = 1024

SPLIT = 16
ROW = VOCAB // SPLIT
N_IDX = SEQ * SPLIT
N_TROWS = NUM_EMB * SPLIT

NC, NS = 2, 16
NW = NC * NS
CHUNK = 64
PER_W = N_IDX // NW


def _table_kern(emb_ref, pw_ref, out_ref):
    out_ref[...] = jax.lax.dot_general(
        emb_ref[...], pw_ref[...],
        (((1,), (1,)), ((), ())),
        preferred_element_type=jnp.float32,
    )


def _make_table(embedding, proj_w):
    return pl.pallas_call(
        _table_kern,
        grid=(VOCAB // V_BLK,),
        in_specs=[
            pl.BlockSpec((NUM_EMB, HIDDEN), lambda j: (0, 0)),
            pl.BlockSpec((V_BLK, HIDDEN), lambda j: (j, 0)),
        ],
        out_specs=pl.BlockSpec((NUM_EMB, V_BLK), lambda j: (0, j)),
        out_shape=jax.ShapeDtypeStruct((NUM_EMB, VOCAB), jnp.float32),
    )(embedding, proj_w)


def _rep_kern(t_ref, out_ref):
    out_ref[0, :, :] = t_ref[...]


def _replicate(table512):
    return pl.pallas_call(
        _rep_kern,
        grid=(NW,),
        in_specs=[pl.BlockSpec((N_TROWS, ROW), lambda j: (0, 0))],
        out_specs=pl.BlockSpec((1, N_TROWS, ROW), lambda j: (j, 0, 0)),
        out_shape=jax.ShapeDtypeStruct((NW, N_TROWS, ROW), jnp.float32),
    )(table512)


def _sc_gather(table_rep, idx):
    mesh = plsc.VectorSubcoreMesh(core_axis_name="c", subcore_axis_name="s")
    n_chunks = PER_W // CHUNK

    @functools.partial(
        pl.kernel,
        mesh=mesh,
        out_type=jax.ShapeDtypeStruct((N_IDX, ROW), jnp.float32),
        scratch_types=[
            pltpu.VMEM((PER_W,), jnp.int32),
            pltpu.VMEM((CHUNK, ROW), jnp.float32),
            pltpu.VMEM((CHUNK, ROW), jnp.float32),
            pltpu.SemaphoreType.DMA,
            pltpu.SemaphoreType.DMA,
            pltpu.SemaphoreType.DMA,
            pltpu.SemaphoreType.DMA,
        ],
    )
    def k(table_hbm, idx_hbm, out_hbm, idx_v, rows0, rows1,
          sem_g0, sem_g1, sem_w0, sem_w1):
        wid = lax.axis_index("s") * NC + lax.axis_index("c")
        base = wid * PER_W
        my_tbl = table_hbm.at[wid]
        rows = (rows0, rows1)
        sem_g = (sem_g0, sem_g1)
        sem_w = (sem_w0, sem_w1)

        pltpu.sync_copy(idx_hbm.at[pl.ds(base, PER_W)], idx_v)

        def islc(c):
            return idx_v.at[pl.ds(c * CHUNK, CHUNK)]

        def g_start(c, b):
            pltpu.async_copy(my_tbl.at[islc(c)], rows[b], sem_g[b])

        def g_wait(c, b):
            pltpu.make_async_copy(my_tbl.at[islc(c)], rows[b], sem_g[b]).wait()

        def w_start(c, b):
            pltpu.async_copy(rows[b], out_hbm.at[pl.ds(base + c * CHUNK, CHUNK)],
                             sem_w[b])

        def w_wait(c, b):
            pltpu.make_async_copy(rows[b],
                                  out_hbm.at[pl.ds(base + c * CHUNK, CHUNK)],
                                  sem_w[b]).wait()

        g_start(0, 0)
        g_start(1, 1)

        @pl.loop(0, n_chunks - 2, step=2)
        def _(i):
            for b in range(2):
                c = i + b
                g_wait(c, b)
                w_start(c, b)
                w_wait(c, b)
                g_start(c + 2, b)

        for b in range(2):
            c = n_chunks - 2 + b
            g_wait(c, b)
            w_start(c, b)
            w_wait(c, b)

    return k(table_rep, idx)


def kernel(input_ids, embedding, proj_w):
    b, s = input_ids.shape
    table = _make_table(embedding, proj_w)
    table_rep = _replicate(table.reshape(N_TROWS, ROW))
    idx = (input_ids.reshape(-1, 1) * SPLIT
           + jnp.arange(SPLIT, dtype=jnp.int32)).reshape(-1)
    out = _sc_gather(table_rep, idx)
    return out.reshape(b, s, VOCAB)


# speedup vs baseline: 1.0310x; 1.0310x over previous
import functools

import jax
import jax.numpy as jnp
from jax import lax
from jax.experimental import pallas as pl
from jax.experimental.pallas import tpu as pltpu
from jax.experimental.pallas import tpu_sc as plsc

HIDDEN = 768
NUM_EMB = 32
VOCAB = 8192
SEQ = 4096
V_BLK = 1024

SPLIT = 2
ROW = VOCAB // SPLIT
N_IDX = SEQ * SPLIT
N_TROWS = NUM_EMB * SPLIT

NC, NS = 2, 16
NW = NC * NS
CHUNK = 8
PER_W = N_IDX // NW


def _table_kern(emb_ref, pw_ref, out_ref):
    out_ref[...] = jax.lax.dot_general(
        emb_ref[...], pw_ref[...],
        (((1,), (1,)), ((), ())),
        preferred_element_type=jnp.float32,
    )


def _make_table(embedding, proj_w):
    return pl.pallas_call(
        _table_kern,
        grid=(VOCAB // V_BLK,),
        in_specs=[
            pl.BlockSpec((NUM_EMB, HIDDEN), lambda j: (0, 0)),
            pl.BlockSpec((V_BLK, HIDDEN), lambda j: (j, 0)),
        ],
        out_specs=pl.BlockSpec((NUM_EMB, V_BLK), lambda j: (0, j)),
        out_shape=jax.ShapeDtypeStruct((NUM_EMB, VOCAB), jnp.float32),
    )(embedding, proj_w)


def _rep_kern(t_ref, out_ref):
    out_ref[0, :, :] = t_ref[...]


def _replicate(table512):
    return pl.pallas_call(
        _rep_kern,
        grid=(NW,),
        in_specs=[pl.BlockSpec((N_TROWS, ROW), lambda j: (0, 0))],
        out_specs=pl.BlockSpec((1, N_TROWS, ROW), lambda j: (j, 0, 0)),
        out_shape=jax.ShapeDtypeStruct((NW, N_TROWS, ROW), jnp.float32),
    )(table512)


def _sc_gather(table_rep, idx):
    mesh = plsc.VectorSubcoreMesh(core_axis_name="c", subcore_axis_name="s")
    n_chunks = PER_W // CHUNK

    @functools.partial(
        pl.kernel,
        mesh=mesh,
        out_type=jax.ShapeDtypeStruct((N_IDX, ROW), jnp.float32),
        scratch_types=[
            pltpu.VMEM((PER_W,), jnp.int32),
            pltpu.VMEM((CHUNK, ROW), jnp.float32),
            pltpu.VMEM((CHUNK, ROW), jnp.float32),
            pltpu.SemaphoreType.DMA,
            pltpu.SemaphoreType.DMA,
            pltpu.SemaphoreType.DMA,
            pltpu.SemaphoreType.DMA,
        ],
    )
    def k(table_hbm, idx_hbm, out_hbm, idx_v, rows0, rows1,
          sem_g0, sem_g1, sem_w0, sem_w1):
        wid = lax.axis_index("s") * NC + lax.axis_index("c")
        base = wid * PER_W
        my_tbl = table_hbm.at[wid]
        rows = (rows0, rows1)
        sem_g = (sem_g0, sem_g1)
        sem_w = (sem_w0, sem_w1)

        pltpu.sync_copy(idx_hbm.at[pl.ds(base, PER_W)], idx_v)

        def islc(c):
            return idx_v.at[pl.ds(c * CHUNK, CHUNK)]

        def g_start(c, b):
            pltpu.async_copy(my_tbl.at[islc(c)], rows[b], sem_g[b])

        def g_wait(c, b):
            pltpu.make_async_copy(my_tbl.at[islc(c)], rows[b], sem_g[b]).wait()

        def w_start(c, b):
            pltpu.async_copy(rows[b], out_hbm.at[pl.ds(base + c * CHUNK, CHUNK)],
                             sem_w[b])

        def w_wait(c, b):
            pltpu.make_async_copy(rows[b],
                                  out_hbm.at[pl.ds(base + c * CHUNK, CHUNK)],
                                  sem_w[b]).wait()

        g_start(0, 0)
        g_start(1, 1)

        @pl.loop(0, n_chunks - 2, step=2)
        def _(i):
            for b in range(2):
                c = i + b
                g_wait(c, b)
                w_start(c, b)
                w_wait(c, b)
                g_start(c + 2, b)

        for b in range(2):
            c = n_chunks - 2 + b
            g_wait(c, b)
            w_start(c, b)
            w_wait(c, b)

    return k(table_rep, idx)


def kernel(input_ids, embedding, proj_w):
    b, s = input_ids.shape
    table = _make_table(embedding, proj_w)
    table_rep = _replicate(table.reshape(N_TROWS, ROW))
    idx = (input_ids.reshape(-1, 1) * SPLIT
           + jnp.arange(SPLIT, dtype=jnp.int32)).reshape(-1)
    out = _sc_gather(table_rep, idx)
    return out.reshape(b, s, VOCAB)


# speedup vs baseline: 5.8809x; 5.7043x over previous
import jax
import jax.numpy as jnp
from jax.experimental import pallas as pl

HIDDEN = 768
NUM_EMB = 32
VOCAB = 8192
V_BLK = 1024


def _kern(ids_ref, emb_ref, pw_ref, out_ref):
    table = jax.lax.dot_general(
        emb_ref[...], pw_ref[...],
        (((1,), (1,)), ((), ())),
        preferred_element_type=jnp.float32,
    )
    ids = ids_ref[0, :]
    s = ids.shape[0]
    onehot = (ids[:, None] == jax.lax.broadcasted_iota(jnp.int32, (s, NUM_EMB), 1)
              ).astype(jnp.bfloat16)
    out_ref[0, :, :] = jnp.dot(onehot, table.astype(jnp.bfloat16),
                               preferred_element_type=jnp.float32)


def kernel(input_ids, embedding, proj_w):
    b, s = input_ids.shape
    return pl.pallas_call(
        _kern,
        grid=(VOCAB // V_BLK,),
        in_specs=[
            pl.BlockSpec((b, s), lambda j: (0, 0)),
            pl.BlockSpec((NUM_EMB, HIDDEN), lambda j: (0, 0)),
            pl.BlockSpec((V_BLK, HIDDEN), lambda j: (j, 0)),
        ],
        out_specs=pl.BlockSpec((b, s, V_BLK), lambda j: (0, 0, j)),
        out_shape=jax.ShapeDtypeStruct((b, s, VOCAB), jnp.float32),
    )(input_ids, embedding, proj_w)
